# baseline (speedup 1.0000x reference)
import functools

import jax
import jax.numpy as jnp
from jax.experimental import pallas as pl
from jax.experimental.pallas import tpu as pltpu

_BLK = 6400


def _col(row_ref):
    return row_ref[...].reshape(32, 1)


def _cell_body(x_ref, ht_ref, ct_ref,
               wit_ref, wft_ref, wct_ref, wot_ref,
               rit_ref, rft_ref, rct_ref, rot_ref,
               cbi_ref, cbf_ref, cbc_ref, cbo_ref,
               bi_ref, bf_ref, bc_ref, bo_ref,
               wci_ref, wcf_ref, wco_ref, lin_wt_ref, lin_b_ref,
               out_ref, h_out_ref, c_out_ref):
    f32 = jnp.float32
    dot_nt = functools.partial(
        jax.lax.dot_general,
        dimension_numbers=(((1,), (1,)), ((), ())),
        preferred_element_type=f32)
    dot_tn = functools.partial(
        jax.lax.dot_general,
        dimension_numbers=(((0,), (0,)), ((), ())),
        preferred_element_type=f32)
    x = x_ref[...]
    ht = ht_ref[...]
    ct = ct_ref[...]

    zi = (dot_nt(wit_ref[...], x)
          + dot_tn(rit_ref[...], ht)
          + (_col(cbi_ref) + _col(bi_ref)) + _col(wci_ref) * ct)
    zf = (dot_nt(wft_ref[...], x)
          + dot_tn(rft_ref[...], ht)
          + (_col(cbf_ref) + _col(bf_ref)) + _col(wcf_ref) * ct)
    zc = (dot_nt(wct_ref[...], x)
          + dot_tn(rct_ref[...], ht)
          + (_col(cbc_ref) + _col(bc_ref)))
    gate_i = jax.nn.sigmoid(zi)
    gate_f = jax.nn.sigmoid(zf)
    gate_t = jnp.tanh(zc)
    c_new = gate_f * ct + gate_i * gate_t
    zo = (dot_nt(wot_ref[...], x)
          + dot_tn(rot_ref[...], ht)
          + (_col(cbo_ref) + _col(bo_ref)) + _col(wco_ref) * c_new)
    gate_o = jax.nn.sigmoid(zo)
    h_new = gate_o * jnp.tanh(c_new)
    out_ref[...] = (jnp.dot(lin_wt_ref[...], h_new, preferred_element_type=f32)
                    + lin_b_ref[...].reshape(1, 1))
    h_out_ref[...] = h_new
    c_out_ref[...] = c_new


def kernel(x, edge_index, edge_weight, h, c, W_i, W_f, W_c, W_o,
           conv_i_w, conv_i_b, conv_f_w, conv_f_b,
           conv_c_w, conv_c_b, conv_o_w, conv_o_b,
           w_c_i, w_c_f, w_c_o,
           b_i, b_f, b_c, b_o,
           lin_w, lin_b):
    del edge_index, edge_weight
    n, f_in = x.shape
    f_out = h.shape[1]

    grid = (pl.cdiv(n, _BLK),)
    x_blk = lambda i: (i, 0)
    col_blk = lambda i: (0, i)
    bcast = lambda i: (0, 0)
    bcast1 = lambda i: (0,)

    out_t, h_new_t, c_new_t = pl.pallas_call(
        _cell_body,
        grid=grid,
        in_specs=[
            pl.BlockSpec((_BLK, f_in), x_blk),
            pl.BlockSpec((f_out, _BLK), col_blk),
            pl.BlockSpec((f_out, _BLK), col_blk),
        ] + [pl.BlockSpec((f_out, f_in), bcast)] * 4
          + [pl.BlockSpec((f_out, f_out), bcast)] * 4
          + [pl.BlockSpec((1, f_out), bcast)] * 4
          + [pl.BlockSpec((1, f_out), bcast)] * 7
          + [
            pl.BlockSpec((1, f_out), bcast),
            pl.BlockSpec((1,), bcast1),
        ],
        out_specs=[
            pl.BlockSpec((1, _BLK), col_blk),
            pl.BlockSpec((f_out, _BLK), col_blk),
            pl.BlockSpec((f_out, _BLK), col_blk),
        ],
        out_shape=[
            jax.ShapeDtypeStruct((1, n), jnp.float32),
            jax.ShapeDtypeStruct((f_out, n), jnp.float32),
            jax.ShapeDtypeStruct((f_out, n), jnp.float32),
        ],
        compiler_params=pltpu.CompilerParams(
            dimension_semantics=("arbitrary",)),
    )(x, h.T, c.T, W_i.T, W_f.T, W_c.T, W_o.T,
      conv_i_w, conv_f_w, conv_c_w, conv_o_w,
      conv_i_b.reshape(1, f_out), conv_f_b.reshape(1, f_out),
      conv_c_b.reshape(1, f_out), conv_o_b.reshape(1, f_out),
      b_i, b_f, b_c, b_o, w_c_i, w_c_f, w_c_o, lin_w.T, lin_b)

    return (out_t.T, h_new_t.T, c_new_t.T)


# speedup vs baseline: 1.0878x; 1.0878x over previous
import functools

import jax
import jax.numpy as jnp
from jax.experimental import pallas as pl
from jax.experimental.pallas import tpu as pltpu

_BLK = 5120


def _col(row_ref):
    return row_ref[...].reshape(32, 1)


def _cell_body(x_ref, ht_ref, ct_ref,
               wit_ref, wft_ref, wct_ref, wot_ref,
               rit_ref, rft_ref, rct_ref, rot_ref,
               cbi_ref, cbf_ref, cbc_ref, cbo_ref,
               bi_ref, bf_ref, bc_ref, bo_ref,
               wci_ref, wcf_ref, wco_ref, lin_wt_ref, lin_b_ref,
               out_ref, h_out_ref, c_out_ref):
    f32 = jnp.float32
    dot_nt = functools.partial(
        jax.lax.dot_general,
        dimension_numbers=(((1,), (1,)), ((), ())),
        preferred_element_type=f32)
    dot_tn = functools.partial(
        jax.lax.dot_general,
        dimension_numbers=(((0,), (0,)), ((), ())),
        preferred_element_type=f32)
    x = x_ref[...]
    ht = ht_ref[...]
    ct = ct_ref[...]

    zi = (dot_nt(wit_ref[...], x)
          + dot_tn(rit_ref[...], ht)
          + (_col(cbi_ref) + _col(bi_ref)) + _col(wci_ref) * ct)
    zf = (dot_nt(wft_ref[...], x)
          + dot_tn(rft_ref[...], ht)
          + (_col(cbf_ref) + _col(bf_ref)) + _col(wcf_ref) * ct)
    zc = (dot_nt(wct_ref[...], x)
          + dot_tn(rct_ref[...], ht)
          + (_col(cbc_ref) + _col(bc_ref)))
    gate_i = jax.nn.sigmoid(zi)
    gate_f = jax.nn.sigmoid(zf)
    gate_t = jnp.tanh(zc)
    c_new = gate_f * ct + gate_i * gate_t
    zo = (dot_nt(wot_ref[...], x)
          + dot_tn(rot_ref[...], ht)
          + (_col(cbo_ref) + _col(bo_ref)) + _col(wco_ref) * c_new)
    gate_o = jax.nn.sigmoid(zo)
    h_new = gate_o * jnp.tanh(c_new)
    out_ref[...] = (jnp.dot(lin_wt_ref[...], h_new, preferred_element_type=f32)
                    + lin_b_ref[...].reshape(1, 1))
    h_out_ref[...] = h_new
    c_out_ref[...] = c_new


def kernel(x, edge_index, edge_weight, h, c, W_i, W_f, W_c, W_o,
           conv_i_w, conv_i_b, conv_f_w, conv_f_b,
           conv_c_w, conv_c_b, conv_o_w, conv_o_b,
           w_c_i, w_c_f, w_c_o,
           b_i, b_f, b_c, b_o,
           lin_w, lin_b):
    del edge_index, edge_weight
    n, f_in = x.shape
    f_out = h.shape[1]

    grid = (pl.cdiv(n, _BLK),)
    x_blk = lambda i: (i, 0)
    col_blk = lambda i: (0, i)
    bcast = lambda i: (0, 0)
    bcast1 = lambda i: (0,)

    out_t, h_new_t, c_new_t = pl.pallas_call(
        _cell_body,
        grid=grid,
        in_specs=[
            pl.BlockSpec((_BLK, f_in), x_blk),
            pl.BlockSpec((f_out, _BLK), col_blk),
            pl.BlockSpec((f_out, _BLK), col_blk),
        ] + [pl.BlockSpec((f_out, f_in), bcast)] * 4
          + [pl.BlockSpec((f_out, f_out), bcast)] * 4
          + [pl.BlockSpec((1, f_out), bcast)] * 4
          + [pl.BlockSpec((1, f_out), bcast)] * 7
          + [
            pl.BlockSpec((1, f_out), bcast),
            pl.BlockSpec((1,), bcast1),
        ],
        out_specs=[
            pl.BlockSpec((1, _BLK), col_blk),
            pl.BlockSpec((f_out, _BLK), col_blk),
            pl.BlockSpec((f_out, _BLK), col_blk),
        ],
        out_shape=[
            jax.ShapeDtypeStruct((1, n), jnp.float32),
            jax.ShapeDtypeStruct((f_out, n), jnp.float32),
            jax.ShapeDtypeStruct((f_out, n), jnp.float32),
        ],
        compiler_params=pltpu.CompilerParams(
            dimension_semantics=("arbitrary",)),
    )(x, h.T, c.T, W_i.T, W_f.T, W_c.T, W_o.T,
      conv_i_w, conv_f_w, conv_c_w, conv_o_w,
      conv_i_b.reshape(1, f_out), conv_f_b.reshape(1, f_out),
      conv_c_b.reshape(1, f_out), conv_o_b.reshape(1, f_out),
      b_i, b_f, b_c, b_o, w_c_i, w_c_f, w_c_o, lin_w.T, lin_b)

    return (out_t.T, h_new_t.T, c_new_t.T)
